# baseline (speedup 1.0000x reference)
import functools

import jax
import jax.numpy as jnp
from jax import lax
from jax.experimental import pallas as pl
from jax.experimental.pallas import tpu as pltpu
from jax.experimental.pallas import tpu_sc as plsc

_EPS = 1e-05
_MAX_NORM = 1.0 - _EPS

_NC = 2
_NS = 16
_NW = _NC * _NS
_LANES = 16
_CHUNK = 128
_PACK = 4


@functools.lru_cache(maxsize=None)
def _make_sc_stats(n, dim):
    assert n % (_NW * _CHUNK) == 0
    assert dim & (dim - 1) == 0
    npw = n // _NW
    nchunk = npw // _CHUNK
    ngroup = _CHUNK // _LANES
    line = _PACK * dim

    mesh = plsc.VectorSubcoreMesh(core_axis_name="c", subcore_axis_name="s")
    out_t = [jax.ShapeDtypeStruct((n,), jnp.float32)] * 4
    scratch = [
        pltpu.VMEM((_CHUNK,), jnp.int32),
        pltpu.VMEM((_CHUNK,), jnp.int32),
        pltpu.VMEM((_CHUNK,), jnp.int32),
        pltpu.VMEM((_CHUNK,), jnp.int32),
        pltpu.VMEM((_CHUNK, line), jnp.float32),
        pltpu.VMEM((_CHUNK, line), jnp.float32),
        pltpu.VMEM((_CHUNK,), jnp.float32),
        pltpu.VMEM((_CHUNK,), jnp.float32),
        pltpu.VMEM((_CHUNK,), jnp.float32),
        pltpu.VMEM((_CHUNK,), jnp.float32),
        pltpu.SemaphoreType.DMA,
    ]

    @functools.partial(pl.kernel, mesh=mesh, out_type=out_t,
                       scratch_types=scratch,
                       compiler_params=pltpu.CompilerParams(
                           needs_layout_passes=False))
    def sc(wq_hbm, x_hbm, y_hbm, jx_hbm, jy_hbm,
           su_hbm, sv_hbm, suv_hbm, dt_hbm,
           idx_x, idx_y, jx_v, jy_v, rows_x, rows_y,
           su_v, sv_v, suv_v, dt_v, sem):
        wid = lax.axis_index("s") * _NC + lax.axis_index("c")
        lanes = lax.iota(jnp.int32, _LANES)

        def chunk_body(c, carry):
            off = (wid * nchunk + c) * _CHUNK
            pltpu.sync_copy(x_hbm.at[pl.ds(off, _CHUNK)], idx_x)
            pltpu.sync_copy(y_hbm.at[pl.ds(off, _CHUNK)], idx_y)
            pltpu.sync_copy(jx_hbm.at[pl.ds(off, _CHUNK)], jx_v)
            pltpu.sync_copy(jy_hbm.at[pl.ds(off, _CHUNK)], jy_v)
            cpx = pltpu.async_copy(wq_hbm.at[jx_v], rows_x, sem)
            cpy = pltpu.async_copy(wq_hbm.at[jy_v], rows_y, sem)
            cpx.wait()
            cpy.wait()

            def group_body(g, gcarry):
                base = g * _LANES
                p = base + lanes
                offx = (idx_x[pl.ds(base, _LANES)] & (_PACK - 1)) * dim
                offy = (idx_y[pl.ds(base, _LANES)] & (_PACK - 1)) * dim
                su = jnp.zeros((_LANES,), jnp.float32)
                sv = jnp.zeros((_LANES,), jnp.float32)
                suv = jnp.zeros((_LANES,), jnp.float32)
                dt = jnp.zeros((_LANES,), jnp.float32)
                for d in range(dim):
                    col = (lanes + d) & (dim - 1)
                    vx = plsc.load_gather(rows_x, [p, offx + col])
                    vy = plsc.load_gather(rows_y, [p, offy + col])
                    su = su + vx * vx
                    sv = sv + vy * vy
                    dt = dt + vx * vy
                    df = vx - vy
                    suv = suv + df * df
                su_v[pl.ds(base, _LANES)] = su
                sv_v[pl.ds(base, _LANES)] = sv
                suv_v[pl.ds(base, _LANES)] = suv
                dt_v[pl.ds(base, _LANES)] = dt
                return gcarry

            lax.fori_loop(0, ngroup, group_body, 0)
            pltpu.sync_copy(su_v, su_hbm.at[pl.ds(off, _CHUNK)])
            pltpu.sync_copy(sv_v, sv_hbm.at[pl.ds(off, _CHUNK)])
            pltpu.sync_copy(suv_v, suv_hbm.at[pl.ds(off, _CHUNK)])
            pltpu.sync_copy(dt_v, dt_hbm.at[pl.ds(off, _CHUNK)])
            return carry

        lax.fori_loop(0, nchunk, chunk_body, 0)

    return sc


def _tc_dist(su_ref, sv_ref, suv_ref, dt_ref, o_ref):
    su = su_ref[...]
    sv = sv_ref[...]
    suv = suv_ref[...]
    dt = dt_ref[...]
    cu = jnp.minimum(1.0, _MAX_NORM / jnp.maximum(jnp.sqrt(su), 1e-12))
    cv = jnp.minimum(1.0, _MAX_NORM / jnp.maximum(jnp.sqrt(sv), 1e-12))
    clamped = jnp.logical_or(cu < 1.0, cv < 1.0)
    suv_eff = jnp.where(
        clamped,
        jnp.maximum(cu * cu * su + cv * cv * sv - 2.0 * cu * cv * dt, 0.0),
        suv)
    norm_u = cu * jnp.sqrt(su)
    norm_v = cv * jnp.sqrt(sv)
    norm_uv = jnp.sqrt(suv_eff)
    d = 1 + 2 * norm_uv ** 2 / ((1 - norm_u ** 2) * (1 - norm_v ** 2))
    o_ref[...] = jnp.log(d + jnp.sqrt((d + 1.0) * (d - 1.0)))


def kernel(x, y, weight):
    b, l = x.shape
    n = b * l
    v, dim = weight.shape
    xf = x.reshape(n).astype(jnp.int32)
    yf = y.reshape(n).astype(jnp.int32)
    w = weight.astype(jnp.float32)
    wq = w.reshape(v // _PACK, _PACK * dim)
    jx = xf // _PACK
    jy = yf // _PACK
    su, sv, suv, dt = _make_sc_stats(n, dim)(wq, xf, yf, jx, jy)
    shape2 = (n // 128, 128)
    dist = pl.pallas_call(
        _tc_dist,
        out_shape=jax.ShapeDtypeStruct(shape2, jnp.float32),
    )(su.reshape(shape2), sv.reshape(shape2),
      suv.reshape(shape2), dt.reshape(shape2))
    return dist.reshape(b, l)


# speedup vs baseline: 1.2267x; 1.2267x over previous
import functools

import jax
import jax.numpy as jnp
from jax import lax
from jax.experimental import pallas as pl
from jax.experimental.pallas import tpu as pltpu
from jax.experimental.pallas import tpu_sc as plsc

_EPS = 1e-05
_MAX_NORM = 1.0 - _EPS

_NC = 2
_NS = 16
_NW = _NC * _NS
_LANES = 16
_CHUNK = 128
_PACK = 4


@functools.lru_cache(maxsize=None)
def _make_sc_stats(n, dim):
    assert n % (_NW * _CHUNK) == 0
    assert dim & (dim - 1) == 0
    npw = n // _NW
    nchunk = npw // _CHUNK
    assert nchunk % 2 == 0
    ngroup = _CHUNK // _LANES
    line = _PACK * dim

    mesh = plsc.VectorSubcoreMesh(core_axis_name="c", subcore_axis_name="s")
    out_t = [jax.ShapeDtypeStruct((n,), jnp.float32)] * 3
    scratch = [
        pltpu.VMEM((npw,), jnp.int32),
        pltpu.VMEM((npw,), jnp.int32),
        pltpu.VMEM((npw,), jnp.int32),
        pltpu.VMEM((npw,), jnp.int32),
        pltpu.VMEM((_CHUNK, line), jnp.float32),
        pltpu.VMEM((_CHUNK, line), jnp.float32),
        pltpu.VMEM((_CHUNK, line), jnp.float32),
        pltpu.VMEM((_CHUNK, line), jnp.float32),
        pltpu.VMEM((npw,), jnp.float32),
        pltpu.VMEM((npw,), jnp.float32),
        pltpu.VMEM((npw,), jnp.float32),
        pltpu.SemaphoreType.DMA,
        pltpu.SemaphoreType.DMA,
    ]

    @functools.partial(pl.kernel, mesh=mesh, out_type=out_t,
                       scratch_types=scratch,
                       compiler_params=pltpu.CompilerParams(
                           needs_layout_passes=False))
    def sc(wq_hbm, x_hbm, y_hbm, jx_hbm, jy_hbm,
           su_hbm, sv_hbm, suv_hbm,
           idx_x, idx_y, jx_v, jy_v,
           rx0, ry0, rx1, ry1, su_v, sv_v, suv_v, sem0, sem1):
        wid = lax.axis_index("s") * _NC + lax.axis_index("c")
        lanes = lax.iota(jnp.int32, _LANES)
        woff = wid * npw

        pltpu.sync_copy(x_hbm.at[pl.ds(woff, npw)], idx_x)
        pltpu.sync_copy(y_hbm.at[pl.ds(woff, npw)], idx_y)
        pltpu.sync_copy(jx_hbm.at[pl.ds(woff, npw)], jx_v)
        pltpu.sync_copy(jy_hbm.at[pl.ds(woff, npw)], jy_v)

        def fire(c, rx, ry, sem):
            pltpu.async_copy(
                wq_hbm.at[jx_v.at[pl.ds(c * _CHUNK, _CHUNK)]], rx, sem)
            pltpu.async_copy(
                wq_hbm.at[jy_v.at[pl.ds(c * _CHUNK, _CHUNK)]], ry, sem)

        def drain(rx, ry, sem):
            pltpu.make_async_copy(
                wq_hbm.at[jx_v.at[pl.ds(0, _CHUNK)]], rx, sem).wait()
            pltpu.make_async_copy(
                wq_hbm.at[jy_v.at[pl.ds(0, _CHUNK)]], ry, sem).wait()

        def compute(c, rx, ry):
            def group_body(g, gc):
                base = g * _LANES
                pos = c * _CHUNK + base
                p = base + lanes
                offx = (idx_x[pl.ds(pos, _LANES)] & (_PACK - 1)) * dim
                offy = (idx_y[pl.ds(pos, _LANES)] & (_PACK - 1)) * dim
                su = jnp.zeros((_LANES,), jnp.float32)
                sv = jnp.zeros((_LANES,), jnp.float32)
                suv = jnp.zeros((_LANES,), jnp.float32)
                for d in range(dim):
                    col = (lanes + d) & (dim - 1)
                    vx = plsc.load_gather(rx, [p, offx + col])
                    vy = plsc.load_gather(ry, [p, offy + col])
                    su = su + vx * vx
                    sv = sv + vy * vy
                    df = vx - vy
                    suv = suv + df * df
                su_v[pl.ds(pos, _LANES)] = su
                sv_v[pl.ds(pos, _LANES)] = sv
                suv_v[pl.ds(pos, _LANES)] = suv
                return gc
            lax.fori_loop(0, ngroup, group_body, 0)

        fire(0, rx0, ry0, sem0)

        def pair_body(t, carry):
            c = 2 * t
            fire(c + 1, rx1, ry1, sem1)
            drain(rx0, ry0, sem0)
            compute(c, rx0, ry0)
            fire(jnp.minimum(c + 2, nchunk - 1), rx0, ry0, sem0)
            drain(rx1, ry1, sem1)
            compute(c + 1, rx1, ry1)
            return carry

        lax.fori_loop(0, nchunk // 2, pair_body, 0)
        drain(rx0, ry0, sem0)

        pltpu.sync_copy(su_v, su_hbm.at[pl.ds(woff, npw)])
        pltpu.sync_copy(sv_v, sv_hbm.at[pl.ds(woff, npw)])
        pltpu.sync_copy(suv_v, suv_hbm.at[pl.ds(woff, npw)])

    return sc


def _tc_dist(su_ref, sv_ref, suv_ref, o_ref):
    su = su_ref[...]
    sv = sv_ref[...]
    suv = suv_ref[...]
    cu = jnp.minimum(1.0, _MAX_NORM / jnp.maximum(jnp.sqrt(su), 1e-12))
    cv = jnp.minimum(1.0, _MAX_NORM / jnp.maximum(jnp.sqrt(sv), 1e-12))
    dt = 0.5 * (su + sv - suv)
    clamped = jnp.logical_or(cu < 1.0, cv < 1.0)
    suv_eff = jnp.where(
        clamped,
        jnp.maximum(cu * cu * su + cv * cv * sv - 2.0 * cu * cv * dt, 0.0),
        suv)
    norm_u = cu * jnp.sqrt(su)
    norm_v = cv * jnp.sqrt(sv)
    norm_uv = jnp.sqrt(suv_eff)
    d = 1 + 2 * norm_uv ** 2 / ((1 - norm_u ** 2) * (1 - norm_v ** 2))
    o_ref[...] = jnp.log(d + jnp.sqrt((d + 1.0) * (d - 1.0)))


def kernel(x, y, weight):
    b, l = x.shape
    n = b * l
    v, dim = weight.shape
    xf = x.reshape(n).astype(jnp.int32)
    yf = y.reshape(n).astype(jnp.int32)
    w = weight.astype(jnp.float32)
    wq = w.reshape(v // _PACK, _PACK * dim)
    jx = xf // _PACK
    jy = yf // _PACK
    su, sv, suv = _make_sc_stats(n, dim)(wq, xf, yf, jx, jy)
    shape2 = (n // 128, 128)
    dist = pl.pallas_call(
        _tc_dist,
        out_shape=jax.ShapeDtypeStruct(shape2, jnp.float32),
    )(su.reshape(shape2), sv.reshape(shape2), suv.reshape(shape2))
    return dist.reshape(b, l)
